# baseline (speedup 1.0000x reference)
import jax
import jax.numpy as jnp
from jax import lax
from jax.experimental import pallas as pl
from jax.experimental.pallas import tpu as pltpu
from jax.experimental.pallas import tpu_sc as plsc

EMBED = 64
SRC_LEN = 200
Q_USED = 36
Q_PAD = 40
BATCH = 1024
SEQ = Q_USED + SRC_LEN
NC, NS = 2, 16
NW = NC * NS
HB = 512
NT = SEQ * (BATCH // HB)
KB = HB // 128
MAX_I = (NT + NW - 1) // NW


def _gather_body(i4_hbm, table_hbm, p_hbm, idx4, buf3, tbuf,
                 sem_g, sem_w):
    wid = lax.axis_index("s") * NC + lax.axis_index("c")
    nt_w = (NT - wid + NW - 1) // NW
    lane = lax.iota(jnp.int32, 16)

    def tile_body(i, carry):
        f = wid + NW * i

        @pl.when(f < NT)
        def _():
            t = f >> 1
            hb = f & 1
            tr = jnp.where(t < Q_USED, t, t + (Q_PAD - Q_USED))
            pltpu.sync_copy(
                i4_hbm.at[tr >> 3, tr & 7, pl.ds(KB * hb, KB)], idx4)
            gds = [
                pltpu.async_copy(table_hbm.at[idx4.at[k]], buf3.at[k], sem_g)
                for k in range(KB)
            ]
            @pl.when(i >= 1)
            def _():
                pltpu.make_async_copy(
                    tbuf, p_hbm.at[t, :, pl.ds(0, HB)], sem_w).wait()
            for gd in gds:
                gd.wait()

            def tr_body(b16, carry2):
                k_vec = jnp.full((16,), b16 >> 3, jnp.int32)
                blo = (b16 & 7) * 16 + lane
                bcol = b16 * 16 + lane
                for c in range(EMBED):
                    c_vec = jnp.full((16,), c, jnp.int32)
                    x = plsc.load_gather(buf3, [k_vec, blo, c_vec])
                    plsc.store_scatter(tbuf, [c_vec, bcol], x)
                return carry2

            lax.fori_loop(0, HB // 16, tr_body, 0)
            pltpu.async_copy(
                tbuf, p_hbm.at[t, :, pl.ds(hb * HB, HB)], sem_w)
        return carry

    lax.fori_loop(0, MAX_I, tile_body, 0)

    @pl.when(nt_w >= 1)
    def _():
        pltpu.make_async_copy(
            tbuf, p_hbm.at[0, :, pl.ds(0, HB)], sem_w).wait()


def kernel(story, question, embed_table):
    idx_all = jnp.concatenate(
        [question[:Q_PAD].astype(jnp.int32), story.astype(jnp.int32)], axis=0)
    i4 = idx_all.reshape((Q_PAD + SRC_LEN) // 8, 8, 8, 128)

    mesh = plsc.VectorSubcoreMesh(
        core_axis_name="c", subcore_axis_name="s",
        num_cores=NC, num_subcores=NS,
    )
    p = pl.kernel(
        _gather_body,
        out_type=jax.ShapeDtypeStruct((SEQ, EMBED, BATCH), jnp.float32),
        mesh=mesh,
        scratch_types=[
            pltpu.VMEM((KB, 128), jnp.int32),
            pltpu.VMEM((KB, 128, EMBED), jnp.float32),
            pltpu.VMEM((EMBED, HB), jnp.float32),
            pltpu.SemaphoreType.DMA,
            pltpu.SemaphoreType.DMA,
        ],
        compiler_params=pltpu.CompilerParams(
            use_tc_tiling_on_sc=False, needs_layout_passes=False),
    )(i4, embed_table)
    return jnp.transpose(p, (2, 0, 1))


# speedup vs baseline: 1.0709x; 1.0709x over previous
import jax
import jax.numpy as jnp
from jax import lax
from jax.experimental import pallas as pl
from jax.experimental.pallas import tpu as pltpu
from jax.experimental.pallas import tpu_sc as plsc

EMBED = 64
SRC_LEN = 200
Q_USED = 36
Q_PAD = 40
BATCH = 1024
SEQ = Q_USED + SRC_LEN
NC, NS = 2, 16
NW = NC * NS
QB = 256
NT = SEQ * (BATCH // QB)
KB = QB // 128
MAX_I = (NT + NW - 1) // NW


def _gather_body(i4_hbm, table_hbm, g_hbm, idx0, idx1, buf0, buf1,
                 sem_g0, sem_g1, sem_w0, sem_w1):
    wid = lax.axis_index("s") * NC + lax.axis_index("c")
    idx2 = (idx0, idx1)
    bufs = (buf0, buf1)
    sem_g = (sem_g0, sem_g1)
    sem_w = (sem_w0, sem_w1)
    gds = [None] * (MAX_I + 1)
    wds = [None] * MAX_I

    def tile_of(i):
        f = wid + NW * i
        t = f >> 2
        qb = f & 3
        tr = jnp.where(t < Q_USED, t, t + (Q_PAD - Q_USED))
        return f, t, qb, tr

    def start_tile(i):
        s = i % 2
        f, t, qb, tr = tile_of(i)
        pltpu.sync_copy(
            i4_hbm.at[tr >> 3, tr & 7, pl.ds(KB * qb, KB)], idx2[s])
        gds[i] = [
            pltpu.async_copy(table_hbm.at[idx2[s].at[k]],
                             bufs[s].at[pl.ds(k * 128, 128)], sem_g[s])
            for k in range(KB)
        ]

    def tile_loop(i, carry):
        f, t, qb, tr = tile_of(i)

        @pl.when(f < NT)
        def _():
            pltpu.sync_copy(
                i4_hbm.at[tr >> 3, tr & 7, pl.ds(KB * qb, KB)], idx0)
            for k in range(KB):
                pltpu.async_copy(table_hbm.at[idx0.at[k]],
                                 buf0.at[pl.ds(k * 128, 128)],
                                 sem_g0).wait()
            pltpu.async_copy(
                buf0, g_hbm.at[t, pl.ds(qb * QB, QB)], sem_w0).wait()
        return carry

    lax.fori_loop(0, MAX_I, tile_loop, 0)


def _tc_transpose_body(g_ref, p_ref):
    x = g_ref[0]
    p_ref[0] = x[:, :EMBED].T


def kernel(story, question, embed_table):
    idx_all = jnp.concatenate(
        [question[:Q_PAD].astype(jnp.int32), story.astype(jnp.int32)], axis=0)
    i4 = idx_all.reshape((Q_PAD + SRC_LEN) // 8, 8, 8, 128)
    table2 = jnp.concatenate([embed_table, embed_table], axis=1)

    mesh = plsc.VectorSubcoreMesh(
        core_axis_name="c", subcore_axis_name="s",
        num_cores=NC, num_subcores=NS,
    )
    g = pl.kernel(
        _gather_body,
        out_type=jax.ShapeDtypeStruct((SEQ, BATCH, 2 * EMBED), jnp.float32),
        mesh=mesh,
        scratch_types=[
            pltpu.VMEM((KB, 128), jnp.int32),
            pltpu.VMEM((KB, 128), jnp.int32),
            pltpu.VMEM((QB, 2 * EMBED), jnp.float32),
            pltpu.VMEM((QB, 2 * EMBED), jnp.float32),
            pltpu.SemaphoreType.DMA,
            pltpu.SemaphoreType.DMA,
            pltpu.SemaphoreType.DMA,
            pltpu.SemaphoreType.DMA,
        ],
        compiler_params=pltpu.CompilerParams(
            use_tc_tiling_on_sc=False, needs_layout_passes=False),
    )(i4, table2)

    p = pl.pallas_call(
        _tc_transpose_body,
        name="tc_transpose",
        grid=(SEQ, BATCH // 512),
        in_specs=[pl.BlockSpec((1, 512, 2 * EMBED), lambda t, h: (t, h, 0))],
        out_specs=pl.BlockSpec((1, EMBED, 512), lambda t, h: (t, 0, h)),
        out_shape=jax.ShapeDtypeStruct((SEQ, EMBED, BATCH), jnp.float32),
    )(g)
    return jnp.transpose(p, (2, 0, 1))


# speedup vs baseline: 1.1807x; 1.1025x over previous
import jax
import jax.numpy as jnp
from jax import lax
from jax.experimental import pallas as pl
from jax.experimental.pallas import tpu as pltpu
from jax.experimental.pallas import tpu_sc as plsc

EMBED = 64
SRC_LEN = 200
Q_USED = 36
Q_PAD = 40
BATCH = 1024
SEQ = Q_USED + SRC_LEN
NC, NS = 2, 16
NW = NC * NS
QB = 256
NT = SEQ * (BATCH // QB)
KB = QB // 128
MAX_I = (NT + NW - 1) // NW


def _gather_body(i4_hbm, table_hbm, g_hbm, idx0, idx1, buf0, buf1,
                 sem_g0, sem_g1, sem_w0, sem_w1):
    wid = lax.axis_index("s") * NC + lax.axis_index("c")

    def tile_of(f):
        t = f >> 2
        qb = f & 3
        tr = jnp.where(t < Q_USED, t, t + (Q_PAD - Q_USED))
        return t, qb, tr

    def pair_loop(i2, carry):
        ta, qa, tra = tile_of(jnp.minimum(wid + NW * (2 * i2), NT - 1))
        tb, qb_, trb = tile_of(jnp.minimum(wid + NW * (2 * i2 + 1), NT - 1))

        pltpu.sync_copy(
            i4_hbm.at[tra >> 3, tra & 7, pl.ds(KB * qa, KB)], idx0)
        ga = [pltpu.async_copy(table_hbm.at[idx0.at[k]],
                               buf0.at[pl.ds(k * 128, 128)], sem_g0)
              for k in range(KB)]
        pltpu.sync_copy(
            i4_hbm.at[trb >> 3, trb & 7, pl.ds(KB * qb_, KB)], idx1)
        gb = [pltpu.async_copy(table_hbm.at[idx1.at[k]],
                               buf1.at[pl.ds(k * 128, 128)], sem_g1)
              for k in range(KB)]
        for gd in ga:
            gd.wait()
        wa = pltpu.async_copy(
            buf0, g_hbm.at[ta, pl.ds(qa * QB, QB)], sem_w0)
        for gd in gb:
            gd.wait()
        wb = pltpu.async_copy(
            buf1, g_hbm.at[tb, pl.ds(qb_ * QB, QB)], sem_w1)
        wa.wait()
        wb.wait()
        return carry

    lax.fori_loop(0, MAX_I // 2, pair_loop, 0)


def _tc_transpose_body(g_ref, p_ref):
    x = g_ref[0]
    p_ref[0] = x[:, :EMBED].T


def kernel(story, question, embed_table):
    idx_all = jnp.concatenate(
        [question[:Q_PAD].astype(jnp.int32), story.astype(jnp.int32)], axis=0)
    i4 = idx_all.reshape((Q_PAD + SRC_LEN) // 8, 8, 8, 128)
    table2 = jnp.concatenate([embed_table, embed_table], axis=1)

    mesh = plsc.VectorSubcoreMesh(
        core_axis_name="c", subcore_axis_name="s",
        num_cores=NC, num_subcores=NS,
    )
    g = pl.kernel(
        _gather_body,
        out_type=jax.ShapeDtypeStruct((SEQ, BATCH, 2 * EMBED), jnp.float32),
        mesh=mesh,
        scratch_types=[
            pltpu.VMEM((KB, 128), jnp.int32),
            pltpu.VMEM((KB, 128), jnp.int32),
            pltpu.VMEM((QB, 2 * EMBED), jnp.float32),
            pltpu.VMEM((QB, 2 * EMBED), jnp.float32),
            pltpu.SemaphoreType.DMA,
            pltpu.SemaphoreType.DMA,
            pltpu.SemaphoreType.DMA,
            pltpu.SemaphoreType.DMA,
        ],
        compiler_params=pltpu.CompilerParams(
            use_tc_tiling_on_sc=False, needs_layout_passes=False),
    )(i4, table2)

    p = pl.pallas_call(
        _tc_transpose_body,
        name="tc_transpose",
        grid=(SEQ, BATCH // 512),
        in_specs=[pl.BlockSpec((1, 512, 2 * EMBED), lambda t, h: (t, h, 0))],
        out_specs=pl.BlockSpec((1, EMBED, 512), lambda t, h: (t, 0, h)),
        out_shape=jax.ShapeDtypeStruct((SEQ, EMBED, BATCH), jnp.float32),
    )(g)
    return jnp.transpose(p, (2, 0, 1))


# speedup vs baseline: 2.0523x; 1.7382x over previous
import jax
import jax.numpy as jnp
from jax import lax
from jax.experimental import pallas as pl
from jax.experimental.pallas import tpu as pltpu
from jax.experimental.pallas import tpu_sc as plsc

EMBED = 64
SRC_LEN = 200
Q_USED = 36
BATCH = 1024
SEQ = Q_USED + SRC_LEN
TOTAL_ROWS = BATCH * SEQ
NC, NS = 2, 16
NW = NC * NS
CHUNK = 128
N_CHUNKS = TOTAL_ROWS // CHUNK
CPW = N_CHUNKS // NW


ROWS_PW = TOTAL_ROWS // NW
BCHUNK = 944
NBCH = ROWS_PW // BCHUNK


def _gather_body(idx_hbm, table_hbm, out_hbm, idx_v, buf0, buf1,
                 sem_g0, sem_g1, sem_w0, sem_w1):
    wid = lax.axis_index("s") * NC + lax.axis_index("c")
    r0 = wid * ROWS_PW
    pltpu.sync_copy(idx_hbm.at[pl.ds(r0, ROWS_PW)], idx_v)

    bufs = (buf0, buf1)
    sem_g = (sem_g0, sem_g1)
    sem_w = (sem_w0, sem_w1)
    gd = [None] * NBCH
    wd = [None] * NBCH

    def start_gather(g):
        gd[g] = pltpu.async_copy(
            table_hbm.at[idx_v.at[pl.ds(g * BCHUNK, BCHUNK)]],
            bufs[g % 2], sem_g[g % 2])

    start_gather(0)
    for g in range(NBCH):
        slot = g % 2
        gd[g].wait()
        if g >= 1:
            wd[g - 1].wait()
        if g < NBCH - 1:
            start_gather(g + 1)
        wd[g] = pltpu.async_copy(
            bufs[slot], out_hbm.at[pl.ds(r0 + g * BCHUNK, BCHUNK)],
            sem_w[slot])
    wd[NBCH - 1].wait()


def kernel(story, question, embed_table):
    idx = jnp.concatenate([question[:Q_USED], story], axis=0)
    idx = idx.astype(jnp.int32).T.reshape(TOTAL_ROWS)

    mesh = plsc.VectorSubcoreMesh(
        core_axis_name="c", subcore_axis_name="s",
        num_cores=NC, num_subcores=NS,
    )
    out = pl.kernel(
        _gather_body,
        out_type=jax.ShapeDtypeStruct((TOTAL_ROWS, EMBED), jnp.float32),
        mesh=mesh,
        scratch_types=[
            pltpu.VMEM((ROWS_PW,), jnp.int32),
            pltpu.VMEM((BCHUNK, EMBED), jnp.float32),
            pltpu.VMEM((BCHUNK, EMBED), jnp.float32),
            pltpu.SemaphoreType.DMA,
            pltpu.SemaphoreType.DMA,
            pltpu.SemaphoreType.DMA,
            pltpu.SemaphoreType.DMA,
        ],
        compiler_params=pltpu.CompilerParams(use_tc_tiling_on_sc=False),
    )(idx, embed_table)
    return out.reshape(BATCH, SEQ, EMBED)
